# Initial kernel scaffold; baseline (speedup 1.0000x reference)
import jax
import jax.numpy as jnp
from jax.experimental import pallas as pl


def kernel(x, W):
    raise NotImplementedError("write your pallas kernel here")


# speedup vs baseline: 1.8697x; 1.8697x over previous
import functools

import jax
import jax.numpy as jnp
from jax import lax
from jax.experimental import pallas as pl
from jax.experimental.pallas import tpu as pltpu
from jax.experimental.pallas import tpu_sc as plsc

VOCAB = 1000000
D = 64
B_TOTAL = 16384 * 50
NC, NS = 2, 16
NW = NC * NS
PER_W = B_TOTAL // NW
GRP = 128
G = PER_W // GRP
K = 8
CHUNKS = G // K

_mesh = plsc.VectorSubcoreMesh(core_axis_name="c", subcore_axis_name="s")


@functools.partial(
    pl.kernel,
    out_type=jax.ShapeDtypeStruct((B_TOTAL, D), jnp.float32),
    mesh=_mesh,
    scratch_types=[
        pltpu.VMEM((G, GRP), jnp.int32),
        pltpu.VMEM((K, GRP, D), jnp.float32),
        pltpu.SemaphoreType.DMA,
        pltpu.SemaphoreType.DMA,
    ],
    compiler_params=pltpu.CompilerParams(use_tc_tiling_on_sc=False),
)
def _embed(idx_hbm, table_hbm, out_hbm, idx_v, rows, gsem, osem):
    wid = lax.axis_index("s") * NC + lax.axis_index("c")
    gbase = wid * G
    obase = wid * PER_W
    pltpu.sync_copy(idx_hbm.at[pl.ds(gbase, G)], idx_v)

    def chunk_body(c, carry):
        g0 = c * K
        for b in range(K):
            pltpu.make_async_copy(
                table_hbm.at[idx_v.at[g0 + b]], rows.at[b], gsem).start()
        for b in range(K):
            pltpu.make_async_copy(
                table_hbm.at[idx_v.at[g0 + b]], rows.at[b], gsem).wait()
        for b in range(K):
            pltpu.make_async_copy(
                rows.at[b],
                out_hbm.at[pl.ds(obase + (g0 + b) * GRP, GRP)], osem).start()
        for b in range(K):
            pltpu.make_async_copy(
                rows.at[b],
                out_hbm.at[pl.ds(obase + (g0 + b) * GRP, GRP)], osem).wait()
        return carry

    lax.fori_loop(0, CHUNKS, chunk_body, 0)


def kernel(x, W):
    idx = x.reshape(B_TOTAL // GRP, GRP).astype(jnp.int32)
    out = _embed(idx, W)
    return out.reshape(x.shape[0], x.shape[1], D)
